# baseline (speedup 1.0000x reference)
import functools

import jax
import jax.numpy as jnp
from jax import lax
from jax.experimental import pallas as pl
from jax.experimental.pallas import tpu as pltpu
from jax.experimental.pallas import tpu_sc as plsc

_B = 4096
_NUM_CLASSES = 64

_THRESH = 0.5
_MARGIN = 0.1
_SCALE_POS = 2.0
_SCALE_NEG = 40.0
_EPS = 1e-5
_THR_CAP = 88.0

_NW = 32
_SC_ROWS = _B
_RPW = _SC_ROWS // _NW
_RCHUNK = 16
_LANES = 16


def _sc_body(sim_hbm, lab_hbm, ps_hbm, es_hbm, tp_hbm, tn_hbm,
             labv, rowbuf, psv, esv, tpv, tnv):
    cid = lax.axis_index("c")
    sid = lax.axis_index("s")
    wid = sid * 2 + cid
    base = wid * _RPW

    pltpu.sync_copy(lab_hbm, labv)

    lanes = lax.iota(jnp.int32, _LANES)
    zero16 = jnp.zeros((_LANES,), jnp.float32)

    def chunk_body(rc, _):
        row0 = base + rc * _RCHUNK
        pltpu.sync_copy(sim_hbm.at[pl.ds(row0, _RCHUNK)], rowbuf)
        labs_self = labv[pl.ds(row0, _RCHUNK)]

        gps, ges, gtp, gtn = zero16, zero16, zero16, zero16
        for j in range(_RCHUNK):
            lab_row = labs_self[j]

            def p1(k, carry):
                vmx, vmn, vxt = carry
                v = rowbuf[j, pl.ds(k * _LANES, _LANES)]
                lab_k = labv[pl.ds(k * _LANES, _LANES)]
                same = lab_k == lab_row
                t = jnp.where(same, v - 2.0, v)
                return (jnp.maximum(vmx, v), jnp.minimum(vmn, t),
                        jnp.maximum(vxt, t))

            neg_fill = zero16 - 3.0
            vmx, vmn, vxt = lax.fori_loop(
                0, _B // _LANES, p1, (neg_fill, zero16 + 3.0, neg_fill))
            row_max = jnp.max(vmx)
            min_pos = jnp.min(vmn) + 2.0
            max_t = jnp.max(vxt)

            max_t16 = max_t + zero16
            max_neg16 = jnp.where(max_t16 >= -0.5, max_t16, -1e30 + zero16)
            min_pos16 = min_pos + zero16
            min_pos16 = jnp.where(min_pos16 < row_max - _EPS,
                                  min_pos16, 1e30 + zero16)
            pos_thr16 = jnp.minimum(row_max - _EPS, max_neg16 + _MARGIN)
            neg_thr16 = min_pos16 - _MARGIN
            u_pos16 = jnp.minimum(-_SCALE_POS * (pos_thr16 - _THRESH),
                                  _THR_CAP + zero16)
            u_neg16 = jnp.minimum(_SCALE_NEG * (neg_thr16 - _THRESH),
                                  _THR_CAP + zero16)

            def p2(k, carry):
                accp, acce = carry
                v = rowbuf[j, pl.ds(k * _LANES, _LANES)]
                lab_k = labv[pl.ds(k * _LANES, _LANES)]
                same = lab_k == lab_row
                a = jnp.where(same, -_SCALE_POS + zero16, _SCALE_NEG + zero16)
                thr = jnp.where(same, u_pos16, u_neg16)
                w = (v - _THRESH) * a - thr
                e = jnp.where(w > 0.0, jnp.exp(w), zero16)
                accp = accp + jnp.where(same, e, zero16)
                return (accp, acce + e)

            accp, acce = lax.fori_loop(0, _B // _LANES, p2, (zero16, zero16))
            psum_raw = jnp.sum(accp)
            esum_raw = jnp.sum(acce)

            sel = lanes == j
            gps = jnp.where(sel, psum_raw + zero16, gps)
            ges = jnp.where(sel, esum_raw + zero16, ges)
            gtp = jnp.where(sel, u_pos16, gtp)
            gtn = jnp.where(sel, u_neg16, gtn)

        off = rc * _RCHUNK
        psv[pl.ds(off, _RCHUNK)] = gps
        esv[pl.ds(off, _RCHUNK)] = ges
        tpv[pl.ds(off, _RCHUNK)] = gtp
        tnv[pl.ds(off, _RCHUNK)] = gtn
        return 0

    lax.fori_loop(0, _RPW // _RCHUNK, chunk_body, 0)

    pltpu.sync_copy(psv, ps_hbm.at[pl.ds(base, _RPW)])
    pltpu.sync_copy(esv, es_hbm.at[pl.ds(base, _RPW)])
    pltpu.sync_copy(tpv, tp_hbm.at[pl.ds(base, _RPW)])
    pltpu.sync_copy(tnv, tn_hbm.at[pl.ds(base, _RPW)])


def _sc_rows(sim_mat, labels):
    f32 = jnp.float32
    out_type = [jax.ShapeDtypeStruct((_SC_ROWS,), f32) for _ in range(4)]
    k = pl.kernel(
        _sc_body,
        out_type=out_type,
        mesh=plsc.VectorSubcoreMesh(core_axis_name="c", subcore_axis_name="s"),
        compiler_params=pltpu.CompilerParams(needs_layout_passes=False),
        scratch_types=[
            pltpu.VMEM((_B,), jnp.int32),
            pltpu.VMEM((_RCHUNK, _B), f32),
            pltpu.VMEM((_RPW,), f32),
            pltpu.VMEM((_RPW,), f32),
            pltpu.VMEM((_RPW,), f32),
            pltpu.VMEM((_RPW,), f32),
        ],
    )
    return k(sim_mat, labels)


def _epi_body(ps_ref, es_ref, tp_ref, tn_ref, lab_ref, out_ref):
    psum_raw = ps_ref[...]
    esum_raw = es_ref[...]
    psum = psum_raw * jnp.exp(tp_ref[...])
    nsum = (esum_raw - psum_raw) * jnp.exp(tn_ref[...])
    per_row = jnp.log1p(psum) / _SCALE_POS + jnp.log1p(nsum) / _SCALE_NEG
    valid = lab_ref[...] != 0
    part = jnp.sum(jnp.where(valid, per_row, 0.0), axis=1, keepdims=True)
    out_ref[...] = part * (1.0 / _B)


def _epilogue(ps, es, tp, tn, labels):
    args = [x.reshape(1, _SC_ROWS) for x in (ps, es, tp, tn)]
    args.append(labels[:_SC_ROWS].reshape(1, _SC_ROWS))
    out = pl.pallas_call(
        _epi_body,
        in_specs=[pl.BlockSpec((1, _SC_ROWS), lambda: (0, 0))] * 5,
        out_specs=pl.BlockSpec((1, 1), lambda: (0, 0)),
        out_shape=jax.ShapeDtypeStruct((1, 1), jnp.float32),
    )(*args)
    return out[0, 0]


def kernel(sim_mat, labels):
    ps, es, tp, tn = _sc_rows(sim_mat, labels)
    return _epilogue(ps, es, tp, tn, labels)


# speedup vs baseline: 4.7892x; 4.7892x over previous
import jax
import jax.numpy as jnp
from jax import lax
from jax.experimental import pallas as pl
from jax.experimental.pallas import tpu as pltpu
from jax.experimental.pallas import tpu_sc as plsc

_B = 4096
_NUM_CLASSES = 64

_THRESH = 0.5
_MARGIN = 0.1
_SCALE_POS = 2.0
_SCALE_NEG = 40.0
_EPS = 1e-5
_LOG2E = 1.4426950408889634
_THR_CAP = 88.0

_SPLIT = 3584
_ROWS = 256

_NW = 32
_SC_ROWS = _B - _SPLIT
_RPW = _SC_ROWS // _NW
_RCHUNK = 16
_LANES = 16


def _tc_body(sim_ref, labr_ref, labc_ref, c_ref, out_ref):
    s = sim_ref[...]
    lab_r = labr_ref[...]
    lab_c = labc_ref[:, :1]
    same = lab_c == lab_r

    t = jnp.where(same, s - 2.0, s)
    row_max = jnp.max(s, axis=1, keepdims=True)
    min_pos = jnp.min(t, axis=1, keepdims=True) + 2.0
    max_t = jnp.max(t, axis=1, keepdims=True)
    max_neg = jnp.where(max_t >= -0.5, max_t, -jnp.inf)
    min_pos = jnp.where(min_pos < row_max - _EPS, min_pos, jnp.inf)

    pos_thr = jnp.minimum(row_max - _EPS, max_neg + _MARGIN)
    neg_thr = min_pos - _MARGIN

    u_pos_thr = jnp.minimum(-_SCALE_POS * (pos_thr - _THRESH), _THR_CAP)
    u_neg_thr = jnp.minimum(_SCALE_NEG * (neg_thr - _THRESH), _THR_CAP)

    a_pos = -_SCALE_POS * _LOG2E
    a_neg = _SCALE_NEG * _LOG2E
    b_pos = (_THRESH * _SCALE_POS - u_pos_thr) * _LOG2E
    b_neg = (-_THRESH * _SCALE_NEG - u_neg_thr) * _LOG2E
    same2 = t < -0.5
    a = jnp.where(same2, a_pos, a_neg)
    b = jnp.where(same2, b_pos, b_neg)
    w = s * a + b
    e = jnp.where(w > 0.0, jnp.exp2(w), 0.0)

    g = jax.lax.dot_general(e, c_ref[...], (((1,), (0,)), ((), ())),
                            preferred_element_type=jnp.float32)
    esum = jnp.sum(g, axis=1, keepdims=True)
    rowhot = lab_c == jax.lax.broadcasted_iota(jnp.int32, (1, _NUM_CLASSES), 1)
    psum_raw = jnp.sum(jnp.where(rowhot, g, 0.0), axis=1, keepdims=True)

    psum = psum_raw * jnp.exp(u_pos_thr)
    nsum = (esum - psum_raw) * jnp.exp(u_neg_thr)

    per_row = jnp.log1p(psum) / _SCALE_POS + jnp.log1p(nsum) / _SCALE_NEG
    valid = lab_c != 0
    part = jnp.sum(jnp.where(valid, per_row, 0.0), axis=0, keepdims=True) * (1.0 / _B)

    @pl.when(pl.program_id(0) == 0)
    def _():
        out_ref[...] = jnp.zeros((1, 1), jnp.float32)

    out_ref[...] += part


def _tc_rows(sim_mat, labels):
    lab_r = labels.reshape(1, _B)
    lab_c = jnp.broadcast_to(labels.reshape(_B, 1), (_B, 128))
    c_mat = (labels.reshape(_B, 1)
             == jnp.arange(_NUM_CLASSES, dtype=jnp.int32).reshape(1, _NUM_CLASSES)
             ).astype(jnp.float32)
    out = pl.pallas_call(
        _tc_body,
        grid=(_SPLIT // _ROWS,),
        in_specs=[
            pl.BlockSpec((_ROWS, _B), lambda i: (i, 0)),
            pl.BlockSpec((1, _B), lambda i: (0, 0)),
            pl.BlockSpec((_ROWS, 128), lambda i: (i, 0)),
            pl.BlockSpec((_B, _NUM_CLASSES), lambda i: (0, 0)),
        ],
        out_specs=pl.BlockSpec((1, 1), lambda i: (0, 0)),
        out_shape=jax.ShapeDtypeStruct((1, 1), jnp.float32),
    )(sim_mat, lab_r, lab_c, c_mat)
    return out


def _sc_body(sim_hbm, lab_hbm, ps_hbm, es_hbm, tp_hbm, tn_hbm,
             labv, rowbuf, psv, esv, tpv, tnv):
    cid = lax.axis_index("c")
    sid = lax.axis_index("s")
    wid = sid * 2 + cid
    base = wid * _RPW

    pltpu.sync_copy(lab_hbm, labv)

    lanes = lax.iota(jnp.int32, _LANES)
    zero16 = jnp.zeros((_LANES,), jnp.float32)

    def chunk_body(rc, _):
        row0 = _SPLIT + base + rc * _RCHUNK
        pltpu.sync_copy(sim_hbm.at[pl.ds(row0, _RCHUNK)], rowbuf)
        labs_self = labv[pl.ds(row0, _RCHUNK)]

        gps, ges, gtp, gtn = zero16, zero16, zero16, zero16
        for j in range(_RCHUNK):
            lab_row = labs_self[j]

            def p1(k, carry):
                vmx, vmn, vxt = carry
                v = rowbuf[j, pl.ds(k * _LANES, _LANES)]
                lab_k = labv[pl.ds(k * _LANES, _LANES)]
                same = lab_k == lab_row
                t = jnp.where(same, v - 2.0, v)
                return (jnp.maximum(vmx, v), jnp.minimum(vmn, t),
                        jnp.maximum(vxt, t))

            neg_fill = zero16 - 3.0
            vmx, vmn, vxt = lax.fori_loop(
                0, _B // _LANES, p1, (neg_fill, zero16 + 3.0, neg_fill))
            row_max = jnp.max(vmx)
            min_pos = jnp.min(vmn) + 2.0
            max_t = jnp.max(vxt)

            max_t16 = max_t + zero16
            max_neg16 = jnp.where(max_t16 >= -0.5, max_t16, -1e30 + zero16)
            min_pos16 = min_pos + zero16
            min_pos16 = jnp.where(min_pos16 < row_max - _EPS,
                                  min_pos16, 1e30 + zero16)
            pos_thr16 = jnp.minimum(row_max - _EPS, max_neg16 + _MARGIN)
            neg_thr16 = min_pos16 - _MARGIN
            u_pos16 = jnp.minimum(-_SCALE_POS * (pos_thr16 - _THRESH),
                                  _THR_CAP + zero16)
            u_neg16 = jnp.minimum(_SCALE_NEG * (neg_thr16 - _THRESH),
                                  _THR_CAP + zero16)

            def p2(k, carry):
                accp, acce = carry
                v = rowbuf[j, pl.ds(k * _LANES, _LANES)]
                lab_k = labv[pl.ds(k * _LANES, _LANES)]
                same = lab_k == lab_row
                aa = jnp.where(same, -_SCALE_POS + zero16, _SCALE_NEG + zero16)
                thr = jnp.where(same, u_pos16, u_neg16)
                w = (v - _THRESH) * aa - thr
                e = jnp.where(w > 0.0, jnp.exp(w), zero16)
                accp = accp + jnp.where(same, e, zero16)
                return (accp, acce + e)

            accp, acce = lax.fori_loop(0, _B // _LANES, p2, (zero16, zero16))
            psum_raw = jnp.sum(accp)
            esum_raw = jnp.sum(acce)

            sel = lanes == j
            gps = jnp.where(sel, psum_raw + zero16, gps)
            ges = jnp.where(sel, esum_raw + zero16, ges)
            gtp = jnp.where(sel, u_pos16, gtp)
            gtn = jnp.where(sel, u_neg16, gtn)

        off = rc * _RCHUNK
        psv[pl.ds(off, _RCHUNK)] = gps
        esv[pl.ds(off, _RCHUNK)] = ges
        tpv[pl.ds(off, _RCHUNK)] = gtp
        tnv[pl.ds(off, _RCHUNK)] = gtn
        return 0

    lax.fori_loop(0, _RPW // _RCHUNK, chunk_body, 0)

    pltpu.sync_copy(psv, ps_hbm.at[pl.ds(base, _RPW)])
    pltpu.sync_copy(esv, es_hbm.at[pl.ds(base, _RPW)])
    pltpu.sync_copy(tpv, tp_hbm.at[pl.ds(base, _RPW)])
    pltpu.sync_copy(tnv, tn_hbm.at[pl.ds(base, _RPW)])


def _sc_rows(sim_mat, labels):
    f32 = jnp.float32
    out_type = [jax.ShapeDtypeStruct((_SC_ROWS,), f32) for _ in range(4)]
    k = pl.kernel(
        _sc_body,
        out_type=out_type,
        mesh=plsc.VectorSubcoreMesh(core_axis_name="c", subcore_axis_name="s"),
        compiler_params=pltpu.CompilerParams(needs_layout_passes=False),
        scratch_types=[
            pltpu.VMEM((_B,), jnp.int32),
            pltpu.VMEM((_RCHUNK, _B), f32),
            pltpu.VMEM((_RPW,), f32),
            pltpu.VMEM((_RPW,), f32),
            pltpu.VMEM((_RPW,), f32),
            pltpu.VMEM((_RPW,), f32),
        ],
    )
    return k(sim_mat, labels)


def _epi_body(ps_ref, es_ref, tp_ref, tn_ref, lab_ref, tc_ref, out_ref):
    psum_raw = ps_ref[...]
    esum_raw = es_ref[...]
    psum = psum_raw * jnp.exp(tp_ref[...])
    nsum = (esum_raw - psum_raw) * jnp.exp(tn_ref[...])
    per_row = jnp.log1p(psum) / _SCALE_POS + jnp.log1p(nsum) / _SCALE_NEG
    valid = lab_ref[...] != 0
    part = jnp.sum(jnp.where(valid, per_row, 0.0), axis=1, keepdims=True)
    out_ref[...] = part * (1.0 / _B) + tc_ref[...]


def _epilogue(ps, es, tp, tn, labels, tc_part):
    args = [x.reshape(1, _SC_ROWS) for x in (ps, es, tp, tn)]
    args.append(labels[_SPLIT:].reshape(1, _SC_ROWS))
    args.append(tc_part)
    out = pl.pallas_call(
        _epi_body,
        in_specs=[pl.BlockSpec((1, _SC_ROWS), lambda: (0, 0))] * 5
        + [pl.BlockSpec((1, 1), lambda: (0, 0))],
        out_specs=pl.BlockSpec((1, 1), lambda: (0, 0)),
        out_shape=jax.ShapeDtypeStruct((1, 1), jnp.float32),
    )(*args)
    return out[0, 0]


def kernel(sim_mat, labels):
    ps, es, tp, tn = _sc_rows(sim_mat, labels)
    tc_part = _tc_rows(sim_mat, labels)
    return _epilogue(ps, es, tp, tn, labels, tc_part)


# speedup vs baseline: 6.7553x; 1.4105x over previous
import jax
import jax.numpy as jnp
from jax.experimental import pallas as pl

_B = 4096
_NUM_CLASSES = 64
_ROWS = 256

_THRESH = 0.5
_MARGIN = 0.1
_SCALE_POS = 2.0
_SCALE_NEG = 40.0
_EPS = 1e-5
_LOG2E = 1.4426950408889634
_THR_CAP = 88.0


def _body(sim_ref, labr_ref, labc_ref, c_ref, out_ref):
    s = sim_ref[...]
    lab_r = labr_ref[...]
    lab_c = labc_ref[:, :1]
    same = lab_c == lab_r

    t = jnp.where(same, s - 2.0, s)
    row_max = jnp.max(s, axis=1, keepdims=True)
    min_pos = jnp.min(t, axis=1, keepdims=True) + 2.0
    max_t = jnp.max(t, axis=1, keepdims=True)
    max_neg = jnp.where(max_t >= -0.5, max_t, -jnp.inf)
    min_pos = jnp.where(min_pos < row_max - _EPS, min_pos, jnp.inf)

    pos_thr = jnp.minimum(row_max - _EPS, max_neg + _MARGIN)
    neg_thr = min_pos - _MARGIN

    u_pos_thr = jnp.minimum(-_SCALE_POS * (pos_thr - _THRESH), _THR_CAP)
    u_neg_thr = jnp.minimum(_SCALE_NEG * (neg_thr - _THRESH), _THR_CAP)

    a_pos = -_SCALE_POS * _LOG2E
    a_neg = _SCALE_NEG * _LOG2E
    b_pos = (_THRESH * _SCALE_POS - u_pos_thr) * _LOG2E
    b_neg = (-_THRESH * _SCALE_NEG - u_neg_thr) * _LOG2E
    same2 = t < -0.5
    a = jnp.where(same2, a_pos, a_neg)
    b = jnp.where(same2, b_pos + 2.0 * a_pos, b_neg)
    w = t * a + b
    e = jnp.where(w > 0.0, jnp.exp2(w), 0.0)

    g = jax.lax.dot_general(e, c_ref[...], (((1,), (0,)), ((), ())),
                            preferred_element_type=jnp.float32)
    esum = jnp.sum(g, axis=1, keepdims=True)
    rowhot = lab_c == jax.lax.broadcasted_iota(jnp.int32, (1, _NUM_CLASSES), 1)
    psum_raw = jnp.sum(jnp.where(rowhot, g, 0.0), axis=1, keepdims=True)

    psum = psum_raw * jnp.exp(u_pos_thr)
    nsum = (esum - psum_raw) * jnp.exp(u_neg_thr)

    per_row = jnp.log1p(psum) / _SCALE_POS + jnp.log1p(nsum) / _SCALE_NEG
    valid = lab_c != 0
    part = jnp.sum(jnp.where(valid, per_row, 0.0), axis=0, keepdims=True) * (1.0 / _B)

    @pl.when(pl.program_id(0) == 0)
    def _():
        out_ref[...] = jnp.zeros((1, 1), jnp.float32)

    out_ref[...] += part


def kernel(sim_mat, labels):
    lab_r = labels.reshape(1, _B)
    lab_c = jnp.broadcast_to(labels.reshape(_B, 1), (_B, 128))
    c_mat = (labels.reshape(_B, 1)
             == jnp.arange(_NUM_CLASSES, dtype=jnp.int32).reshape(1, _NUM_CLASSES)
             ).astype(jnp.float32)
    out = pl.pallas_call(
        _body,
        grid=(_B // _ROWS,),
        in_specs=[
            pl.BlockSpec((_ROWS, _B), lambda i: (i, 0)),
            pl.BlockSpec((1, _B), lambda i: (0, 0)),
            pl.BlockSpec((_ROWS, 128), lambda i: (i, 0)),
            pl.BlockSpec((_B, _NUM_CLASSES), lambda i: (0, 0)),
        ],
        out_specs=pl.BlockSpec((1, 1), lambda i: (0, 0)),
        out_shape=jax.ShapeDtypeStruct((1, 1), jnp.float32),
    )(sim_mat, lab_r, lab_c, c_mat)
    return out[0, 0]
